# baseline (speedup 1.0000x reference)
import jax
import jax.numpy as jnp
from jax import lax
from jax.experimental import pallas as pl
from jax.experimental.pallas import tpu as pltpu


def _batched_vecmat(lhs, s):
    return lax.dot_general(
        lhs, s, (((2,), (1,)), ((0,), (0,))),
        preferred_element_type=jnp.float32)


def _cell_kernel(x_ref, w_ref, s0_ref, da_ref, ba_ref,
                 out_ref, sf_ref, kvqa_scr, s_scr):
    C, BH, N = out_ref.shape
    D = x_ref.shape[2]
    tc = pl.program_id(1)

    @pl.when(tc == 0)
    def _init():
        s_scr[...] = s0_ref[...]

    xb = x_ref[...].reshape(C * BH, D)
    kvqa_scr[...] = jnp.dot(
        xb, w_ref[...], preferred_element_type=jnp.float32
    ).reshape(C, BH, 4 * N)

    da = da_ref[...]
    ba = ba_ref[...]

    k0 = kvqa_scr[0][:, 0:N]
    r0 = _batched_vecmat(k0[:, None, :], s_scr[...])[:, 0, :]
    G = BH // 2

    def half_step(i, ip1, r, lo):
        kvqa = kvqa_scr[i, lo:lo + G]
        k_t = kvqa[:, 0:N]
        v_t = kvqa[:, N:2 * N]
        q_t = kvqa[:, 2 * N:3 * N]
        ax_t = kvqa[:, 3 * N:4 * N]
        alpha = jax.nn.sigmoid(ax_t + da * r + ba)
        w = (1.0 - alpha) * v_t
        S_new = (alpha[:, None, :] * s_scr[lo:lo + G]
                 + k_t[:, :, None] * w[:, None, :])
        s_scr[lo:lo + G] = S_new
        k_next = kvqa_scr[ip1, lo:lo + G, 0:N]
        lhs = jnp.stack([k_next, q_t], axis=1)
        P = _batched_vecmat(lhs, S_new)
        h = P[:, 1, :]
        out_ref[i, lo:lo + G] = h * h * jax.nn.sigmoid(h)
        return P[:, 0, :]

    def step(i, r):
        rA, rB = r
        ip1 = jnp.where(i + 1 < C, i + 1, 0)
        rA2 = half_step(i, ip1, rA, 0)
        rB2 = half_step(i, ip1, rB, G)
        return (rA2, rB2)

    lax.fori_loop(0, C, step, (r0[0:G], r0[G:BH]))

    @pl.when(tc == pl.num_programs(1) - 1)
    def _fin():
        sf_ref[...] = s_scr[...]


def kernel(x, S0, W_k, W_v, W_q, W_alpha, d_alpha, b_alpha):
    T, B, D = x.shape
    N = W_k.shape[0]
    NC = 2
    BH = B // NC
    C = 32
    assert T % C == 0 and B % NC == 0

    W_all = jnp.concatenate(
        [W_k.T, W_v.T, W_q.T, W_alpha.T], axis=1)
    da = d_alpha.reshape(1, N)
    ba = b_alpha.reshape(1, N)
    S0_T = S0.swapaxes(1, 2)

    out, SfT = pl.pallas_call(
        _cell_kernel,
        grid=(NC, T // C),
        in_specs=[
            pl.BlockSpec((C, BH, D), lambda c, t: (t, c, 0)),
            pl.BlockSpec((D, 4 * N), lambda c, t: (0, 0)),
            pl.BlockSpec((BH, N, N), lambda c, t: (c, 0, 0)),
            pl.BlockSpec((1, N), lambda c, t: (0, 0)),
            pl.BlockSpec((1, N), lambda c, t: (0, 0)),
        ],
        out_specs=[
            pl.BlockSpec((C, BH, N), lambda c, t: (t, c, 0)),
            pl.BlockSpec((BH, N, N), lambda c, t: (c, 0, 0)),
        ],
        out_shape=[
            jax.ShapeDtypeStruct((T, B, N), jnp.float32),
            jax.ShapeDtypeStruct((B, N, N), jnp.float32),
        ],
        scratch_shapes=[
            pltpu.VMEM((C, BH, 4 * N), jnp.float32),
            pltpu.VMEM((BH, N, N), jnp.float32),
        ],
        compiler_params=pltpu.CompilerParams(
            dimension_semantics=("parallel", "arbitrary"),
        ),
    )(x, W_all, S0_T, da, ba)
    return out, SfT.swapaxes(1, 2)


# speedup vs baseline: 1.2069x; 1.2069x over previous
import jax
import jax.numpy as jnp
from jax import lax
from jax.experimental import pallas as pl
from jax.experimental.pallas import tpu as pltpu


def _batched_vecmat(lhs, s):
    return lax.dot_general(
        lhs, s, (((2,), (1,)), ((0,), (0,))),
        preferred_element_type=jnp.float32)


def _cell_kernel(x_ref, w_ref, s0_ref, da_ref, ba_ref,
                 out_ref, sf_ref, kvqa_scr, *s_scrs):
    C, B, N = out_ref.shape
    D = x_ref.shape[2]
    NG = len(s_scrs)
    G = B // NG
    tc = pl.program_id(0)

    @pl.when(tc == 0)
    def _init():
        for g in range(NG):
            s_scrs[g][...] = s0_ref[g * G:(g + 1) * G]

    xb = x_ref[...].reshape(C * B, D)
    kvqa_scr[...] = jnp.dot(
        xb, w_ref[...], preferred_element_type=jnp.float32
    ).reshape(C, B, 4 * N)

    da = da_ref[...]
    ba = ba_ref[...]

    k0 = kvqa_scr[0][:, 0:N]
    r0 = tuple(
        _batched_vecmat(k0[g * G:(g + 1) * G, None, :], s_scrs[g][...])[:, 0, :]
        for g in range(NG))

    def chain_step(i, ip1, r, lo, s_scr):
        kvqa = kvqa_scr[i, lo:lo + G]
        k_t = kvqa[:, 0:N]
        v_t = kvqa[:, N:2 * N]
        q_t = kvqa[:, 2 * N:3 * N]
        ax_t = kvqa[:, 3 * N:4 * N]
        alpha = jax.nn.sigmoid(ax_t + da * r + ba)
        w = (1.0 - alpha) * v_t
        S_new = (alpha[:, None, :] * s_scr[...]
                 + k_t[:, :, None] * w[:, None, :])
        s_scr[...] = S_new
        k_next = kvqa_scr[ip1, lo:lo + G, 0:N]
        lhs = jnp.stack([k_next, q_t], axis=1)
        P = _batched_vecmat(lhs, S_new)
        h = P[:, 1, :]
        out_ref[i, lo:lo + G] = h * h * jax.nn.sigmoid(h)
        return P[:, 0, :]

    def step(i, r):
        ip1 = jnp.where(i + 1 < C, i + 1, 0)
        return tuple(
            chain_step(i, ip1, r[g], g * G, s_scrs[g]) for g in range(NG))

    lax.fori_loop(0, C, step, r0)

    @pl.when(tc == pl.num_programs(0) - 1)
    def _fin():
        for g in range(NG):
            sf_ref[g * G:(g + 1) * G] = s_scrs[g][...]


def kernel(x, S0, W_k, W_v, W_q, W_alpha, d_alpha, b_alpha):
    T, B, D = x.shape
    N = W_k.shape[0]
    C = 32
    NG = 4
    assert T % C == 0 and B % NG == 0

    W_all = jnp.concatenate(
        [W_k.T, W_v.T, W_q.T, W_alpha.T], axis=1)
    da = d_alpha.reshape(1, N)
    ba = b_alpha.reshape(1, N)
    S0_T = S0.swapaxes(1, 2)

    out, SfT = pl.pallas_call(
        _cell_kernel,
        grid=(T // C,),
        in_specs=[
            pl.BlockSpec((C, B, D), lambda t: (t, 0, 0)),
            pl.BlockSpec((D, 4 * N), lambda t: (0, 0)),
            pl.BlockSpec((B, N, N), lambda t: (0, 0, 0)),
            pl.BlockSpec((1, N), lambda t: (0, 0)),
            pl.BlockSpec((1, N), lambda t: (0, 0)),
        ],
        out_specs=[
            pl.BlockSpec((C, B, N), lambda t: (t, 0, 0)),
            pl.BlockSpec((B, N, N), lambda t: (0, 0, 0)),
        ],
        out_shape=[
            jax.ShapeDtypeStruct((T, B, N), jnp.float32),
            jax.ShapeDtypeStruct((B, N, N), jnp.float32),
        ],
        scratch_shapes=(
            [pltpu.VMEM((C, B, 4 * N), jnp.float32)]
            + [pltpu.VMEM((B // NG, N, N), jnp.float32) for _ in range(NG)]
        ),
        compiler_params=pltpu.CompilerParams(
            dimension_semantics=("arbitrary",),
        ),
    )(x, W_all, S0_T, da, ba)
    return out, SfT.swapaxes(1, 2)


# speedup vs baseline: 2.3252x; 1.9266x over previous
import jax
import jax.numpy as jnp
from jax import lax
from jax.experimental import pallas as pl
from jax.experimental.pallas import tpu as pltpu

CSUB = 8


def _cell_kernel(x_ref, w_ref, s0_ref, da_ref, ba_ref,
                 out_ref, sf_ref):
    _, B, N = out_ref.shape
    D = x_ref.shape[2]
    tc = pl.program_id(0)

    @pl.when(tc == 0)
    def _init():
        sf_ref[...] = s0_ref[...]

    xb = x_ref[...].reshape(CSUB * B, D)
    kvqa = jnp.dot(
        xb, w_ref[...], preferred_element_type=jnp.float32
    ).reshape(CSUB, B, 4 * N)

    da = da_ref[...]
    ba = ba_ref[...]

    ks = [kvqa[t, :, 0:N] for t in range(CSUB)]
    vs = [kvqa[t, :, N:2 * N] for t in range(CSUB)]
    qs = [kvqa[t, :, 2 * N:3 * N] for t in range(CSUB)]
    axs = [kvqa[t, :, 3 * N:4 * N] for t in range(CSUB)]

    ck = {(t, s): jnp.sum(ks[t] * ks[s], axis=1, keepdims=True)
          for t in range(CSUB) for s in range(t)}
    cq = {(t, s): jnp.sum(qs[t] * ks[s], axis=1, keepdims=True)
          for t in range(CSUB) for s in range(t + 1)}

    lhs_kq = jnp.stack(ks + qs, axis=1)
    S_base = sf_ref[...]
    PR = lax.dot_general(
        lhs_kq, S_base, (((2,), (1,)), ((0,), (0,))),
        preferred_element_type=jnp.float32)

    G = []
    cum = None
    for t in range(CSUB):
        r = PR[:, t, :] if cum is None else PR[:, t, :] * cum
        for s in range(t):
            r = r + ck[(t, s)] * G[s]
        alpha = jax.nn.sigmoid(axs[t] + da * r + ba)
        w = (1.0 - alpha) * vs[t]
        G = [g * alpha for g in G] + [w]
        cum = alpha if cum is None else cum * alpha
        h = PR[:, CSUB + t, :] * cum
        for s in range(t + 1):
            h = h + cq[(t, s)] * G[s]
        out_ref[t] = h * h * jax.nn.sigmoid(h)

    K_stack = lhs_kq[:, 0:CSUB, :]
    G_stack = jnp.stack(G, axis=1)
    FoldM = lax.dot_general(
        K_stack, G_stack, (((1,), (1,)), ((0,), (0,))),
        preferred_element_type=jnp.float32)
    sf_ref[...] = sf_ref[...] * cum[:, None, :] + FoldM


def kernel(x, S0, W_k, W_v, W_q, W_alpha, d_alpha, b_alpha):
    T, B, D = x.shape
    N = W_k.shape[0]
    assert T % CSUB == 0

    W_all = jnp.concatenate(
        [W_k.T, W_v.T, W_q.T, W_alpha.T], axis=1)
    da = d_alpha.reshape(1, N)
    ba = b_alpha.reshape(1, N)
    S0_T = S0.swapaxes(1, 2)

    out, SfT = pl.pallas_call(
        _cell_kernel,
        grid=(T // CSUB,),
        in_specs=[
            pl.BlockSpec((CSUB, B, D), lambda t: (t, 0, 0)),
            pl.BlockSpec((D, 4 * N), lambda t: (0, 0)),
            pl.BlockSpec((B, N, N), lambda t: (0, 0, 0)),
            pl.BlockSpec((1, N), lambda t: (0, 0)),
            pl.BlockSpec((1, N), lambda t: (0, 0)),
        ],
        out_specs=[
            pl.BlockSpec((CSUB, B, N), lambda t: (t, 0, 0)),
            pl.BlockSpec((B, N, N), lambda t: (0, 0, 0)),
        ],
        out_shape=[
            jax.ShapeDtypeStruct((T, B, N), jnp.float32),
            jax.ShapeDtypeStruct((B, N, N), jnp.float32),
        ],
        compiler_params=pltpu.CompilerParams(
            dimension_semantics=("arbitrary",),
        ),
    )(x, W_all, S0_T, da, ba)
    return out, SfT.swapaxes(1, 2)
